# baseline (speedup 1.0000x reference)
import dataclasses
import functools

import jax
import jax.numpy as jnp
from jax import lax
from jax.experimental import pallas as pl
from jax.experimental.pallas import tpu as pltpu
from jax.experimental.pallas import tpu_sc as plsc

N_ROWS = 320000
N_FEAT = 128
N_SEG = 10000
N_WORKERS = 32
SEG_PER_W = 320
N_SEG_PAD = N_WORKERS * SEG_PER_W
CHUNK = 256
N_LANE = 16


def _bounds_tc_kernel(ids_ref, out_ref):
    ids = ids_ref[...]
    for k in range(N_WORKERS + 1):
        out_ref[k] = jnp.sum((ids < SEG_PER_W * k).astype(jnp.int32))
    for k in range(N_WORKERS + 1, 48):
        out_ref[k] = N_ROWS


def _row_bounds(ids32):
    return pl.pallas_call(
        _bounds_tc_kernel,
        out_shape=jax.ShapeDtypeStruct((48,), jnp.int32),
        out_specs=pl.BlockSpec(memory_space=pltpu.SMEM),
    )(ids32.reshape(N_ROWS // 128, 128))


def _sc_segment_max(x, ids32, row_bounds):
    mesh = plsc.VectorSubcoreMesh(core_axis_name="c", subcore_axis_name="s")
    cp = pltpu.CompilerParams()
    if "needs_layout_passes" in pltpu.CompilerParams.__dataclass_fields__:
        cp = dataclasses.replace(cp, needs_layout_passes=False)

    @functools.partial(
        pl.kernel,
        out_type=jax.ShapeDtypeStruct((N_SEG_PAD, N_FEAT), jnp.float32),
        mesh=mesh,
        compiler_params=cp,
        scratch_types=[
            pltpu.VMEM((CHUNK, N_FEAT), jnp.float32),
            pltpu.VMEM((CHUNK, N_FEAT), jnp.float32),
            pltpu.VMEM((CHUNK + 16,), jnp.int32),
            pltpu.VMEM((CHUNK + 16,), jnp.int32),
            pltpu.VMEM((48,), jnp.int32),
            pltpu.VMEM((SEG_PER_W, N_FEAT), jnp.float32),
            pltpu.SemaphoreType.DMA,
            pltpu.SemaphoreType.DMA,
        ],
    )
    def sc_kernel(x_hbm, ids_hbm, rb_hbm, out_hbm,
                  xbuf0, xbuf1, idbuf0, idbuf1, rbbuf, acc, sem0, sem1):
        w = lax.axis_index("c") * 16 + lax.axis_index("s")
        slo = w * SEG_PER_W

        pltpu.sync_copy(rb_hbm, rbbuf)
        rbv = rbbuf[pl.ds(w, 16)]
        r_lo = rbv[0]
        r_hi = rbv[1]

        neg_inf = jnp.full((N_LANE,), -jnp.inf, jnp.float32)
        NJ = N_FEAT // N_LANE

        @pl.loop(0, SEG_PER_W)
        def _(r):
            for j in range(NJ):
                acc[r, pl.ds(j * N_LANE, N_LANE)] = neg_inf

        c_lo = lax.div(r_lo, CHUNK)
        c_hi = lax.div(r_hi + (CHUNK - 1), CHUNK)

        def step(xbuf, r, nli, carry):
            li = carry[0]
            accs = carry[1:]
            xs = [xbuf[r, pl.ds(j * N_LANE, N_LANE)] for j in range(NJ)]

            boundary = nli != li

            @pl.when(boundary & (li >= 0))
            def _():
                for j in range(NJ):
                    sl = pl.ds(j * N_LANE, N_LANE)
                    acc[li, sl] = jnp.maximum(acc[li, sl], accs[j])

            new_accs = tuple(
                jnp.where(boundary, xs[j], jnp.maximum(accs[j], xs[j]))
                for j in range(NJ)
            )
            return (nli,) + new_accs

        def make_row_body(xbuf, idbuf):
            def row_body(r, carry):
                nli = idbuf[pl.ds(r, 16)][0] - slo
                return step(xbuf, r, nli, carry)
            return row_body

        def start(c, xbuf, idbuf, sem):
            base = c * CHUNK
            pltpu.async_copy(x_hbm.at[pl.ds(base, CHUNK)], xbuf, sem)
            pltpu.async_copy(
                ids_hbm.at[pl.ds(base, CHUNK)], idbuf.at[pl.ds(0, CHUNK)], sem)

        def wait(c, xbuf, idbuf, sem):
            base = c * CHUNK
            pltpu.make_async_copy(x_hbm.at[pl.ds(base, CHUNK)], xbuf, sem).wait()
            pltpu.make_async_copy(
                ids_hbm.at[pl.ds(base, CHUNK)], idbuf.at[pl.ds(0, CHUNK)], sem).wait()

        UNROLL = 16

        def process(c, xbuf, idbuf, carry):
            base = c * CHUNK
            lo_rel = jnp.maximum(r_lo - base, 0)
            hi_rel = jnp.maximum(jnp.minimum(r_hi - base, CHUNK), lo_rel)
            row_body = make_row_body(xbuf, idbuf)
            lo_al = jnp.minimum(
                jnp.bitwise_and(lo_rel + (UNROLL - 1), -UNROLL), hi_rel)
            carry = lax.fori_loop(lo_rel, lo_al, row_body, carry)
            nq = lax.div(hi_rel - lo_al, UNROLL)

            def group_body(q, carry):
                r0 = lo_al + q * UNROLL
                idv = idbuf[pl.ds(r0, 16)]
                li = carry[0]
                first = idv[0]
                allsame = jnp.all(idv == jnp.full((16,), first, jnp.int32))
                uniform = allsame & (first - slo == li)
                n8 = jnp.where(uniform, 2, 0)

                def fast8(b, accs):
                    rb = r0 + b * 8
                    for i in range(8):
                        xs = [xbuf[rb + i, pl.ds(j * N_LANE, N_LANE)]
                              for j in range(NJ)]
                        accs = tuple(jnp.maximum(accs[j], xs[j])
                                     for j in range(NJ))
                    return accs

                accs = lax.fori_loop(0, n8, fast8, carry[1:])
                carry = (li,) + accs
                slow_start = jnp.where(uniform, r0 + UNROLL, r0)
                return lax.fori_loop(slow_start, r0 + UNROLL, row_body, carry)

            carry = lax.fori_loop(0, nq, group_body, carry)
            return lax.fori_loop(lo_al + nq * UNROLL, hi_rel, row_body, carry)

        n = c_hi - c_lo

        @pl.when(n > 0)
        def _():
            start(c_lo, xbuf0, idbuf0, sem0)

        def pair_body(kk, carry):
            c0 = c_lo + 2 * kk
            wait(c0, xbuf0, idbuf0, sem0)

            @pl.when(c0 + 1 < c_hi)
            def _():
                start(c0 + 1, xbuf1, idbuf1, sem1)

            carry = process(c0, xbuf0, idbuf0, carry)

            @pl.when(c0 + 1 < c_hi)
            def _():
                wait(c0 + 1, xbuf1, idbuf1, sem1)

            @pl.when(c0 + 2 < c_hi)
            def _():
                start(c0 + 2, xbuf0, idbuf0, sem0)

            carry = process(c0 + 1, xbuf1, idbuf1, carry)
            return carry

        init = (jnp.int32(-1),) + tuple(neg_inf for _ in range(NJ))
        final = lax.fori_loop(0, lax.div(n + 1, 2), pair_body, init)
        fli = final[0]

        @pl.when(fli >= 0)
        def _():
            for j in range(NJ):
                sl = pl.ds(j * N_LANE, N_LANE)
                acc[fli, sl] = jnp.maximum(acc[fli, sl], final[1 + j])

        pltpu.sync_copy(acc, out_hbm.at[pl.ds(slo, SEG_PER_W)])

    return sc_kernel(x, ids32, row_bounds)


def kernel(x, segment_ids):
    ids32 = segment_ids.astype(jnp.int32)
    row_bounds = _row_bounds(ids32)
    out_pad = _sc_segment_max(x, ids32, row_bounds)
    return out_pad[:N_SEG]


# speedup vs baseline: 1.3064x; 1.3064x over previous
import dataclasses
import functools

import jax
import jax.numpy as jnp
from jax import lax
from jax.experimental import pallas as pl
from jax.experimental.pallas import tpu as pltpu
from jax.experimental.pallas import tpu_sc as plsc

N_ROWS = 320000
N_FEAT = 128
N_SEG = 10000
N_WORKERS = 32
SEG_PER_W = 320
N_SEG_PAD = N_WORKERS * SEG_PER_W
CHUNK = 256
N_LANE = 16


def _bounds_tc_kernel(ids_ref, out_ref):
    ids = ids_ref[...]
    for k in range(N_WORKERS + 1):
        out_ref[k] = jnp.sum((ids < SEG_PER_W * k).astype(jnp.int32))
    for k in range(N_WORKERS + 1, 48):
        out_ref[k] = N_ROWS


def _row_bounds(ids32):
    return pl.pallas_call(
        _bounds_tc_kernel,
        out_shape=jax.ShapeDtypeStruct((48,), jnp.int32),
        out_specs=pl.BlockSpec(memory_space=pltpu.SMEM),
    )(ids32.reshape(N_ROWS // 128, 128))


def _sc_segment_max(x, ids32, row_bounds):
    mesh = plsc.VectorSubcoreMesh(core_axis_name="c", subcore_axis_name="s")
    cp = pltpu.CompilerParams()
    if "needs_layout_passes" in pltpu.CompilerParams.__dataclass_fields__:
        cp = dataclasses.replace(cp, needs_layout_passes=False)

    @functools.partial(
        pl.kernel,
        out_type=jax.ShapeDtypeStruct((N_SEG_PAD, N_FEAT), jnp.float32),
        mesh=mesh,
        compiler_params=cp,
        scratch_types=[
            pltpu.VMEM((CHUNK, N_FEAT), jnp.float32),
            pltpu.VMEM((CHUNK, N_FEAT), jnp.float32),
            pltpu.VMEM((CHUNK + 16,), jnp.int32),
            pltpu.VMEM((CHUNK + 16,), jnp.int32),
            pltpu.VMEM((48,), jnp.int32),
            pltpu.VMEM((SEG_PER_W, N_FEAT), jnp.float32),
            pltpu.SemaphoreType.DMA,
            pltpu.SemaphoreType.DMA,
        ],
    )
    def sc_kernel(x_hbm, ids_hbm, rb_hbm, out_hbm,
                  xbuf0, xbuf1, idbuf0, idbuf1, rbbuf, acc, sem0, sem1):
        w = lax.axis_index("c") * 16 + lax.axis_index("s")
        slo = w * SEG_PER_W

        pltpu.sync_copy(rb_hbm, rbbuf)
        rbv = rbbuf[pl.ds(w, 16)]
        r_lo = rbv[0]
        r_hi = rbv[1]

        neg_inf = jnp.full((N_LANE,), -jnp.inf, jnp.float32)
        NJ = N_FEAT // N_LANE

        @pl.loop(0, SEG_PER_W)
        def _(r):
            for j in range(NJ):
                acc[r, pl.ds(j * N_LANE, N_LANE)] = neg_inf

        c_lo = lax.div(r_lo, CHUNK)
        c_hi = lax.div(r_hi + (CHUNK - 1), CHUNK)

        def step(xbuf, r, nli, carry):
            li = carry[0]
            accs = carry[1:]
            xs = [xbuf[r, pl.ds(j * N_LANE, N_LANE)] for j in range(NJ)]

            boundary = nli != li

            @pl.when(boundary & (li >= 0))
            def _():
                for j in range(NJ):
                    sl = pl.ds(j * N_LANE, N_LANE)
                    acc[li, sl] = jnp.maximum(acc[li, sl], accs[j])

            new_accs = tuple(
                jnp.where(boundary, xs[j], jnp.maximum(accs[j], xs[j]))
                for j in range(NJ)
            )
            return (nli,) + new_accs

        def make_row_body(xbuf, idbuf):
            def row_body(r, carry):
                nli = idbuf[pl.ds(r, 16)][0] - slo
                return step(xbuf, r, nli, carry)
            return row_body

        def start(c, xbuf, idbuf, sem):
            base = c * CHUNK
            pltpu.async_copy(x_hbm.at[pl.ds(base, CHUNK)], xbuf, sem)
            pltpu.async_copy(
                ids_hbm.at[pl.ds(base, CHUNK)], idbuf.at[pl.ds(0, CHUNK)], sem)

        def wait(c, xbuf, idbuf, sem):
            base = c * CHUNK
            pltpu.make_async_copy(x_hbm.at[pl.ds(base, CHUNK)], xbuf, sem).wait()
            pltpu.make_async_copy(
                ids_hbm.at[pl.ds(base, CHUNK)], idbuf.at[pl.ds(0, CHUNK)], sem).wait()

        UNROLL = 16

        def process(c, xbuf, idbuf, carry):
            base = c * CHUNK
            lo_rel = jnp.maximum(r_lo - base, 0)
            hi_rel = jnp.maximum(jnp.minimum(r_hi - base, CHUNK), lo_rel)
            row_body = make_row_body(xbuf, idbuf)
            lo_al = jnp.minimum(
                jnp.bitwise_and(lo_rel + (UNROLL - 1), -UNROLL), hi_rel)
            carry = lax.fori_loop(lo_rel, lo_al, row_body, carry)
            nq = lax.div(hi_rel - lo_al, UNROLL)

            def group_body(q, carry):
                r0 = lo_al + q * UNROLL
                idv = idbuf[pl.ds(r0, 16)]
                li = carry[0]
                first = idv[0]
                allsame = jnp.all(idv == jnp.full((16,), first, jnp.int32))
                uniform = allsame & (first - slo == li)
                n8 = jnp.where(uniform, 2, 0)

                def fast8(b, accs):
                    rb = r0 + b * 8
                    for i in range(8):
                        xs = [xbuf[rb + i, pl.ds(j * N_LANE, N_LANE)]
                              for j in range(NJ)]
                        accs = tuple(jnp.maximum(accs[j], xs[j])
                                     for j in range(NJ))
                    return accs

                accs = lax.fori_loop(0, n8, fast8, carry[1:])
                carry = (li,) + accs

                def slow16(_, carry):
                    for i in range(UNROLL):
                        carry = step(xbuf, r0 + i, idv[i] - slo, carry)
                    return carry

                nslow = jnp.where(uniform, 0, 1)
                return lax.fori_loop(0, nslow, slow16, carry)

            carry = lax.fori_loop(0, nq, group_body, carry)
            return lax.fori_loop(lo_al + nq * UNROLL, hi_rel, row_body, carry)

        n = c_hi - c_lo

        @pl.when(n > 0)
        def _():
            start(c_lo, xbuf0, idbuf0, sem0)

        def pair_body(kk, carry):
            c0 = c_lo + 2 * kk
            wait(c0, xbuf0, idbuf0, sem0)

            @pl.when(c0 + 1 < c_hi)
            def _():
                start(c0 + 1, xbuf1, idbuf1, sem1)

            carry = process(c0, xbuf0, idbuf0, carry)

            @pl.when(c0 + 1 < c_hi)
            def _():
                wait(c0 + 1, xbuf1, idbuf1, sem1)

            @pl.when(c0 + 2 < c_hi)
            def _():
                start(c0 + 2, xbuf0, idbuf0, sem0)

            carry = process(c0 + 1, xbuf1, idbuf1, carry)
            return carry

        init = (jnp.int32(-1),) + tuple(neg_inf for _ in range(NJ))
        final = lax.fori_loop(0, lax.div(n + 1, 2), pair_body, init)
        fli = final[0]

        @pl.when(fli >= 0)
        def _():
            for j in range(NJ):
                sl = pl.ds(j * N_LANE, N_LANE)
                acc[fli, sl] = jnp.maximum(acc[fli, sl], final[1 + j])

        pltpu.sync_copy(acc, out_hbm.at[pl.ds(slo, SEG_PER_W)])

    return sc_kernel(x, ids32, row_bounds)


def kernel(x, segment_ids):
    ids32 = segment_ids.astype(jnp.int32)
    row_bounds = _row_bounds(ids32)
    out_pad = _sc_segment_max(x, ids32, row_bounds)
    return out_pad[:N_SEG]


# speedup vs baseline: 1.4961x; 1.1452x over previous
import dataclasses
import functools

import jax
import jax.numpy as jnp
from jax import lax
from jax.experimental import pallas as pl
from jax.experimental.pallas import tpu as pltpu
from jax.experimental.pallas import tpu_sc as plsc

N_ROWS = 320000
N_FEAT = 128
N_SEG = 10000
N_WORKERS = 32
SEG_PER_W = 320
N_SEG_PAD = N_WORKERS * SEG_PER_W
CHUNK = 256
N_LANE = 16


def _bounds_tc_kernel(ids_ref, out_ref):
    ids = ids_ref[...]
    for k in range(N_WORKERS + 1):
        out_ref[k] = jnp.sum((ids < SEG_PER_W * k).astype(jnp.int32))
    for k in range(N_WORKERS + 1, 48):
        out_ref[k] = N_ROWS


def _row_bounds(ids32):
    return pl.pallas_call(
        _bounds_tc_kernel,
        out_shape=jax.ShapeDtypeStruct((48,), jnp.int32),
        out_specs=pl.BlockSpec(memory_space=pltpu.SMEM),
    )(ids32.reshape(N_ROWS // 128, 128))


def _sc_segment_max(x, ids32, row_bounds):
    mesh = plsc.VectorSubcoreMesh(core_axis_name="c", subcore_axis_name="s")
    cp = pltpu.CompilerParams()
    if "needs_layout_passes" in pltpu.CompilerParams.__dataclass_fields__:
        cp = dataclasses.replace(cp, needs_layout_passes=False)

    @functools.partial(
        pl.kernel,
        out_type=jax.ShapeDtypeStruct((N_SEG_PAD, N_FEAT), jnp.float32),
        mesh=mesh,
        compiler_params=cp,
        scratch_types=[
            pltpu.VMEM((CHUNK, N_FEAT), jnp.float32),
            pltpu.VMEM((CHUNK, N_FEAT), jnp.float32),
            pltpu.VMEM((CHUNK + 16,), jnp.int32),
            pltpu.VMEM((CHUNK + 16,), jnp.int32),
            pltpu.VMEM((48,), jnp.int32),
            pltpu.VMEM((SEG_PER_W, N_FEAT), jnp.float32),
            pltpu.SemaphoreType.DMA,
            pltpu.SemaphoreType.DMA,
        ],
    )
    def sc_kernel(x_hbm, ids_hbm, rb_hbm, out_hbm,
                  xbuf0, xbuf1, idbuf0, idbuf1, rbbuf, acc, sem0, sem1):
        w = lax.axis_index("c") * 16 + lax.axis_index("s")
        slo = w * SEG_PER_W

        pltpu.sync_copy(rb_hbm, rbbuf)
        rbv = rbbuf[pl.ds(w, 16)]
        r_lo = rbv[0]
        r_hi = rbv[1]

        neg_inf = jnp.full((N_LANE,), -jnp.inf, jnp.float32)
        NJ = N_FEAT // N_LANE

        @pl.loop(0, SEG_PER_W)
        def _(r):
            for j in range(NJ):
                acc[r, pl.ds(j * N_LANE, N_LANE)] = neg_inf

        c_lo = lax.div(r_lo, CHUNK)
        c_hi = lax.div(r_hi + (CHUNK - 1), CHUNK)

        def step(xbuf, r, nli, carry):
            li = carry[0]
            accs = carry[1:]
            xs = [xbuf[r, pl.ds(j * N_LANE, N_LANE)] for j in range(NJ)]

            boundary = nli != li

            @pl.when(boundary & (li >= 0))
            def _():
                for j in range(NJ):
                    sl = pl.ds(j * N_LANE, N_LANE)
                    acc[li, sl] = jnp.maximum(acc[li, sl], accs[j])

            new_accs = tuple(
                jnp.where(boundary, xs[j], jnp.maximum(accs[j], xs[j]))
                for j in range(NJ)
            )
            return (nli,) + new_accs

        def make_row_body(xbuf, idbuf):
            def row_body(r, carry):
                nli = idbuf[pl.ds(r, 16)][0] - slo
                return step(xbuf, r, nli, carry)
            return row_body

        def start(c, xbuf, idbuf, sem):
            base = c * CHUNK
            pltpu.async_copy(x_hbm.at[pl.ds(base, CHUNK)], xbuf, sem)
            pltpu.async_copy(
                ids_hbm.at[pl.ds(base, CHUNK)], idbuf.at[pl.ds(0, CHUNK)], sem)

        def wait(c, xbuf, idbuf, sem):
            base = c * CHUNK
            pltpu.make_async_copy(x_hbm.at[pl.ds(base, CHUNK)], xbuf, sem).wait()
            pltpu.make_async_copy(
                ids_hbm.at[pl.ds(base, CHUNK)], idbuf.at[pl.ds(0, CHUNK)], sem).wait()

        UNROLL = 16

        def process(c, xbuf, idbuf, carry):
            base = c * CHUNK
            lo_rel = jnp.maximum(r_lo - base, 0)
            hi_rel = jnp.maximum(jnp.minimum(r_hi - base, CHUNK), lo_rel)
            row_body = make_row_body(xbuf, idbuf)
            lo_al = jnp.minimum(
                jnp.bitwise_and(lo_rel + (UNROLL - 1), -UNROLL), hi_rel)
            carry = lax.fori_loop(lo_rel, lo_al, row_body, carry)
            nq = lax.div(hi_rel - lo_al, UNROLL)

            iota16 = lax.iota(jnp.int32, 16)
            shift_idx = jnp.maximum(iota16 - 1, 0)

            def plain_row(r, accs):
                xs = [xbuf[r, pl.ds(j * N_LANE, N_LANE)] for j in range(NJ)]
                return tuple(jnp.maximum(accs[j], xs[j]) for j in range(NJ))

            def group_body(q, carry):
                r0 = lo_al + q * UNROLL
                idv = idbuf[pl.ds(r0, 16)]
                li = carry[0]
                accs = carry[1:]
                first = idv[0]
                prev = lax.gather(
                    idv, shift_idx[:, None],
                    lax.GatherDimensionNumbers(
                        offset_dims=(), collapsed_slice_dims=(0,),
                        start_index_map=(0,)),
                    slice_sizes=(1,),
                    mode=lax.GatherScatterMode.PROMISE_IN_BOUNDS)
                m_int = (idv != prev) & (iota16 > 0)
                b_int = plsc.all_reduce_population_count(m_int)[0]
                m0 = (first - slo) != li
                nb = b_int + jnp.where(m0, 1, 0)

                n8 = jnp.where(nb == 0, 2, 0)

                def fast8(b, accs):
                    rb = r0 + b * 8
                    for i in range(8):
                        accs = plain_row(rb + i, accs)
                    return accs

                accs = lax.fori_loop(0, n8, fast8, accs)

                p = jnp.where(m0, 0, plsc.all_reduce_ffs(m_int)[0])
                e1 = jnp.where(nb == 1, p, 0)
                accs = lax.fori_loop(r0, r0 + e1, plain_row, accs)

                @pl.when((nb == 1) & (li >= 0))
                def _():
                    for j in range(NJ):
                        sl = pl.ds(j * N_LANE, N_LANE)
                        acc[li, sl] = jnp.maximum(acc[li, sl], accs[j])

                accs = tuple(jnp.where(nb == 1, neg_inf, accs[j])
                             for j in range(NJ))
                s2 = jnp.where(nb == 1, p, UNROLL)
                accs = lax.fori_loop(r0 + s2, r0 + UNROLL, plain_row, accs)
                li = jnp.where(nb == 1, idv[15] - slo, li)

                def slow16(_, carry):
                    for i in range(UNROLL):
                        carry = step(xbuf, r0 + i, idv[i] - slo, carry)
                    return carry

                nslow = jnp.where(nb >= 2, 1, 0)
                return lax.fori_loop(0, nslow, slow16, (li,) + accs)

            carry = lax.fori_loop(0, nq, group_body, carry)
            return lax.fori_loop(lo_al + nq * UNROLL, hi_rel, row_body, carry)

        n = c_hi - c_lo

        @pl.when(n > 0)
        def _():
            start(c_lo, xbuf0, idbuf0, sem0)

        def pair_body(kk, carry):
            c0 = c_lo + 2 * kk
            wait(c0, xbuf0, idbuf0, sem0)

            @pl.when(c0 + 1 < c_hi)
            def _():
                start(c0 + 1, xbuf1, idbuf1, sem1)

            carry = process(c0, xbuf0, idbuf0, carry)

            @pl.when(c0 + 1 < c_hi)
            def _():
                wait(c0 + 1, xbuf1, idbuf1, sem1)

            @pl.when(c0 + 2 < c_hi)
            def _():
                start(c0 + 2, xbuf0, idbuf0, sem0)

            carry = process(c0 + 1, xbuf1, idbuf1, carry)
            return carry

        init = (jnp.int32(-1),) + tuple(neg_inf for _ in range(NJ))
        final = lax.fori_loop(0, lax.div(n + 1, 2), pair_body, init)
        fli = final[0]

        @pl.when(fli >= 0)
        def _():
            for j in range(NJ):
                sl = pl.ds(j * N_LANE, N_LANE)
                acc[fli, sl] = jnp.maximum(acc[fli, sl], final[1 + j])

        pltpu.sync_copy(acc, out_hbm.at[pl.ds(slo, SEG_PER_W)])

    return sc_kernel(x, ids32, row_bounds)


def kernel(x, segment_ids):
    ids32 = segment_ids.astype(jnp.int32)
    row_bounds = _row_bounds(ids32)
    out_pad = _sc_segment_max(x, ids32, row_bounds)
    return out_pad[:N_SEG]


# speedup vs baseline: 1.4992x; 1.0021x over previous
import dataclasses
import functools

import jax
import jax.numpy as jnp
from jax import lax
from jax.experimental import pallas as pl
from jax.experimental.pallas import tpu as pltpu
from jax.experimental.pallas import tpu_sc as plsc

N_ROWS = 320000
N_FEAT = 128
N_SEG = 10000
N_WORKERS = 32
SEG_PER_W = 320
N_SEG_PAD = N_WORKERS * SEG_PER_W
CHUNK = 320
N_LANE = 16


def _bounds_tc_kernel(ids_ref, out_ref):
    ids = ids_ref[...]
    for k in range(N_WORKERS + 1):
        out_ref[k] = jnp.sum((ids < SEG_PER_W * k).astype(jnp.int32))
    for k in range(N_WORKERS + 1, 48):
        out_ref[k] = N_ROWS


def _row_bounds(ids32):
    return pl.pallas_call(
        _bounds_tc_kernel,
        out_shape=jax.ShapeDtypeStruct((48,), jnp.int32),
        out_specs=pl.BlockSpec(memory_space=pltpu.SMEM),
    )(ids32.reshape(N_ROWS // 128, 128))


def _sc_segment_max(x, ids32, row_bounds):
    mesh = plsc.VectorSubcoreMesh(core_axis_name="c", subcore_axis_name="s")
    cp = pltpu.CompilerParams()
    if "needs_layout_passes" in pltpu.CompilerParams.__dataclass_fields__:
        cp = dataclasses.replace(cp, needs_layout_passes=False)

    @functools.partial(
        pl.kernel,
        out_type=jax.ShapeDtypeStruct((N_SEG_PAD, N_FEAT), jnp.float32),
        mesh=mesh,
        compiler_params=cp,
        scratch_types=[
            pltpu.VMEM((CHUNK, N_FEAT), jnp.float32),
            pltpu.VMEM((CHUNK, N_FEAT), jnp.float32),
            pltpu.VMEM((CHUNK + 16,), jnp.int32),
            pltpu.VMEM((CHUNK + 16,), jnp.int32),
            pltpu.VMEM((48,), jnp.int32),
            pltpu.VMEM((SEG_PER_W, N_FEAT), jnp.float32),
            pltpu.SemaphoreType.DMA,
            pltpu.SemaphoreType.DMA,
        ],
    )
    def sc_kernel(x_hbm, ids_hbm, rb_hbm, out_hbm,
                  xbuf0, xbuf1, idbuf0, idbuf1, rbbuf, acc, sem0, sem1):
        w = lax.axis_index("c") * 16 + lax.axis_index("s")
        slo = w * SEG_PER_W

        pltpu.sync_copy(rb_hbm, rbbuf)
        rbv = rbbuf[pl.ds(w, 16)]
        r_lo = rbv[0]
        r_hi = rbv[1]

        neg_inf = jnp.full((N_LANE,), -jnp.inf, jnp.float32)
        NJ = N_FEAT // N_LANE

        @pl.loop(0, SEG_PER_W)
        def _(r):
            for j in range(NJ):
                acc[r, pl.ds(j * N_LANE, N_LANE)] = neg_inf

        c_lo = lax.div(r_lo, CHUNK)
        c_hi = lax.div(r_hi + (CHUNK - 1), CHUNK)

        def step(xbuf, r, nli, carry):
            li = carry[0]
            accs = carry[1:]
            xs = [xbuf[r, pl.ds(j * N_LANE, N_LANE)] for j in range(NJ)]

            boundary = nli != li

            @pl.when(boundary & (li >= 0))
            def _():
                for j in range(NJ):
                    sl = pl.ds(j * N_LANE, N_LANE)
                    acc[li, sl] = jnp.maximum(acc[li, sl], accs[j])

            new_accs = tuple(
                jnp.where(boundary, xs[j], jnp.maximum(accs[j], xs[j]))
                for j in range(NJ)
            )
            return (nli,) + new_accs

        def make_row_body(xbuf, idbuf):
            def row_body(r, carry):
                nli = idbuf[pl.ds(r, 16)][0] - slo
                return step(xbuf, r, nli, carry)
            return row_body

        def start(c, xbuf, idbuf, sem):
            base = c * CHUNK
            pltpu.async_copy(x_hbm.at[pl.ds(base, CHUNK)], xbuf, sem)
            pltpu.async_copy(
                ids_hbm.at[pl.ds(base, CHUNK)], idbuf.at[pl.ds(0, CHUNK)], sem)

        def wait(c, xbuf, idbuf, sem):
            base = c * CHUNK
            pltpu.make_async_copy(x_hbm.at[pl.ds(base, CHUNK)], xbuf, sem).wait()
            pltpu.make_async_copy(
                ids_hbm.at[pl.ds(base, CHUNK)], idbuf.at[pl.ds(0, CHUNK)], sem).wait()

        UNROLL = 16

        def process(c, xbuf, idbuf, carry):
            base = c * CHUNK
            lo_rel = jnp.maximum(r_lo - base, 0)
            hi_rel = jnp.maximum(jnp.minimum(r_hi - base, CHUNK), lo_rel)
            row_body = make_row_body(xbuf, idbuf)
            lo_al = jnp.minimum(
                jnp.bitwise_and(lo_rel + (UNROLL - 1), -UNROLL), hi_rel)
            carry = lax.fori_loop(lo_rel, lo_al, row_body, carry)
            nq = lax.div(hi_rel - lo_al, UNROLL)

            iota16 = lax.iota(jnp.int32, 16)
            shift_idx = jnp.maximum(iota16 - 1, 0)

            def plain_row(r, accs):
                xs = [xbuf[r, pl.ds(j * N_LANE, N_LANE)] for j in range(NJ)]
                return tuple(jnp.maximum(accs[j], xs[j]) for j in range(NJ))

            def group_body(q, carry):
                r0 = lo_al + q * UNROLL
                idv = idbuf[pl.ds(r0, 16)]
                li = carry[0]
                accs = carry[1:]
                first = idv[0]
                prev = lax.gather(
                    idv, shift_idx[:, None],
                    lax.GatherDimensionNumbers(
                        offset_dims=(), collapsed_slice_dims=(0,),
                        start_index_map=(0,)),
                    slice_sizes=(1,),
                    mode=lax.GatherScatterMode.PROMISE_IN_BOUNDS)
                m_int = (idv != prev) & (iota16 > 0)
                b_int = plsc.all_reduce_population_count(m_int)[0]
                m0 = (first - slo) != li
                nb = b_int + jnp.where(m0, 1, 0)

                n8 = jnp.where(nb == 0, 2, 0)

                def fast8(b, accs):
                    rb = r0 + b * 8
                    for i in range(8):
                        accs = plain_row(rb + i, accs)
                    return accs

                accs = lax.fori_loop(0, n8, fast8, accs)

                p = jnp.where(m0, 0, plsc.all_reduce_ffs(m_int)[0])
                e1 = jnp.where(nb == 1, p, 0)
                accs = lax.fori_loop(r0, r0 + e1, plain_row, accs)

                @pl.when((nb == 1) & (li >= 0))
                def _():
                    for j in range(NJ):
                        sl = pl.ds(j * N_LANE, N_LANE)
                        acc[li, sl] = jnp.maximum(acc[li, sl], accs[j])

                accs = tuple(jnp.where(nb == 1, neg_inf, accs[j])
                             for j in range(NJ))
                s2 = jnp.where(nb == 1, p, UNROLL)
                accs = lax.fori_loop(r0 + s2, r0 + UNROLL, plain_row, accs)
                li = jnp.where(nb == 1, idv[15] - slo, li)

                def slow16(_, carry):
                    for i in range(UNROLL):
                        carry = step(xbuf, r0 + i, idv[i] - slo, carry)
                    return carry

                nslow = jnp.where(nb >= 2, 1, 0)
                return lax.fori_loop(0, nslow, slow16, (li,) + accs)

            carry = lax.fori_loop(0, nq, group_body, carry)
            return lax.fori_loop(lo_al + nq * UNROLL, hi_rel, row_body, carry)

        n = c_hi - c_lo

        @pl.when(n > 0)
        def _():
            start(c_lo, xbuf0, idbuf0, sem0)

        def pair_body(kk, carry):
            c0 = c_lo + 2 * kk
            wait(c0, xbuf0, idbuf0, sem0)

            @pl.when(c0 + 1 < c_hi)
            def _():
                start(c0 + 1, xbuf1, idbuf1, sem1)

            carry = process(c0, xbuf0, idbuf0, carry)

            @pl.when(c0 + 1 < c_hi)
            def _():
                wait(c0 + 1, xbuf1, idbuf1, sem1)

            @pl.when(c0 + 2 < c_hi)
            def _():
                start(c0 + 2, xbuf0, idbuf0, sem0)

            carry = process(c0 + 1, xbuf1, idbuf1, carry)
            return carry

        init = (jnp.int32(-1),) + tuple(neg_inf for _ in range(NJ))
        final = lax.fori_loop(0, lax.div(n + 1, 2), pair_body, init)
        fli = final[0]

        @pl.when(fli >= 0)
        def _():
            for j in range(NJ):
                sl = pl.ds(j * N_LANE, N_LANE)
                acc[fli, sl] = jnp.maximum(acc[fli, sl], final[1 + j])

        pltpu.sync_copy(acc, out_hbm.at[pl.ds(slo, SEG_PER_W)])

    return sc_kernel(x, ids32, row_bounds)


def kernel(x, segment_ids):
    ids32 = segment_ids.astype(jnp.int32)
    row_bounds = _row_bounds(ids32)
    out_pad = _sc_segment_max(x, ids32, row_bounds)
    return out_pad[:N_SEG]


# speedup vs baseline: 1.5153x; 1.0107x over previous
import dataclasses
import functools

import jax
import jax.numpy as jnp
from jax import lax
from jax.experimental import pallas as pl
from jax.experimental.pallas import tpu as pltpu
from jax.experimental.pallas import tpu_sc as plsc

N_ROWS = 320000
N_FEAT = 128
N_SEG = 10000
N_WORKERS = 32
SEG_PER_W = 320
N_SEG_PAD = N_WORKERS * SEG_PER_W
CHUNK = 320
N_LANE = 16


def _bounds_tc_kernel(ids_ref, out_ref):
    ids = ids_ref[...]
    for k in range(N_WORKERS + 1):
        out_ref[k] = jnp.sum((ids < SEG_PER_W * k).astype(jnp.int32))
    for k in range(N_WORKERS + 1, 48):
        out_ref[k] = N_ROWS


def _row_bounds(ids32):
    return pl.pallas_call(
        _bounds_tc_kernel,
        out_shape=jax.ShapeDtypeStruct((48,), jnp.int32),
        out_specs=pl.BlockSpec(memory_space=pltpu.SMEM),
    )(ids32.reshape(N_ROWS // 128, 128))


def _sc_segment_max(x, ids32, row_bounds):
    mesh = plsc.VectorSubcoreMesh(core_axis_name="c", subcore_axis_name="s")
    cp = pltpu.CompilerParams()
    if "needs_layout_passes" in pltpu.CompilerParams.__dataclass_fields__:
        cp = dataclasses.replace(cp, needs_layout_passes=False)

    @functools.partial(
        pl.kernel,
        out_type=jax.ShapeDtypeStruct((N_SEG_PAD, N_FEAT), jnp.float32),
        mesh=mesh,
        compiler_params=cp,
        scratch_types=[
            pltpu.VMEM((CHUNK, N_FEAT), jnp.float32),
            pltpu.VMEM((CHUNK, N_FEAT), jnp.float32),
            pltpu.VMEM((CHUNK + 16,), jnp.int32),
            pltpu.VMEM((CHUNK + 16,), jnp.int32),
            pltpu.VMEM((48,), jnp.int32),
            pltpu.VMEM((SEG_PER_W, N_FEAT), jnp.float32),
            pltpu.SemaphoreType.DMA,
            pltpu.SemaphoreType.DMA,
        ],
    )
    def sc_kernel(x_hbm, ids_hbm, rb_hbm, out_hbm,
                  xbuf0, xbuf1, idbuf0, idbuf1, rbbuf, acc, sem0, sem1):
        w = lax.axis_index("c") * 16 + lax.axis_index("s")
        slo = w * SEG_PER_W

        pltpu.sync_copy(rb_hbm, rbbuf)
        rbv = rbbuf[pl.ds(w, 16)]
        r_lo = rbv[0]
        r_hi = rbv[1]

        neg_inf = jnp.full((N_LANE,), -jnp.inf, jnp.float32)
        NJ = N_FEAT // N_LANE

        c_lo = lax.div(r_lo, CHUNK)
        c_hi = lax.div(r_hi + (CHUNK - 1), CHUNK)

        def step(xbuf, r, nli, carry):
            li = carry[0]
            accs = carry[1:]
            xs = [xbuf[r, pl.ds(j * N_LANE, N_LANE)] for j in range(NJ)]

            boundary = nli != li

            @pl.when(boundary & (li >= 0))
            def _():
                for j in range(NJ):
                    sl = pl.ds(j * N_LANE, N_LANE)
                    acc[li, sl] = jnp.maximum(acc[li, sl], accs[j])

            new_accs = tuple(
                jnp.where(boundary, xs[j], jnp.maximum(accs[j], xs[j]))
                for j in range(NJ)
            )
            return (nli,) + new_accs

        def make_row_body(xbuf, idbuf):
            def row_body(r, carry):
                nli = idbuf[pl.ds(r, 16)][0] - slo
                return step(xbuf, r, nli, carry)
            return row_body

        def start(c, xbuf, idbuf, sem):
            base = c * CHUNK
            pltpu.async_copy(x_hbm.at[pl.ds(base, CHUNK)], xbuf, sem)
            pltpu.async_copy(
                ids_hbm.at[pl.ds(base, CHUNK)], idbuf.at[pl.ds(0, CHUNK)], sem)

        def wait(c, xbuf, idbuf, sem):
            base = c * CHUNK
            pltpu.make_async_copy(x_hbm.at[pl.ds(base, CHUNK)], xbuf, sem).wait()
            pltpu.make_async_copy(
                ids_hbm.at[pl.ds(base, CHUNK)], idbuf.at[pl.ds(0, CHUNK)], sem).wait()

        UNROLL = 16

        def process(c, xbuf, idbuf, carry):
            base = c * CHUNK
            lo_rel = jnp.maximum(r_lo - base, 0)
            hi_rel = jnp.maximum(jnp.minimum(r_hi - base, CHUNK), lo_rel)
            row_body = make_row_body(xbuf, idbuf)
            lo_al = jnp.minimum(
                jnp.bitwise_and(lo_rel + (UNROLL - 1), -UNROLL), hi_rel)
            carry = lax.fori_loop(lo_rel, lo_al, row_body, carry)
            nq = lax.div(hi_rel - lo_al, UNROLL)

            iota16 = lax.iota(jnp.int32, 16)
            shift_idx = jnp.maximum(iota16 - 1, 0)

            def plain_row(r, accs):
                xs = [xbuf[r, pl.ds(j * N_LANE, N_LANE)] for j in range(NJ)]
                return tuple(jnp.maximum(accs[j], xs[j]) for j in range(NJ))

            def group_body(q, carry):
                r0 = lo_al + q * UNROLL
                idv = idbuf[pl.ds(r0, 16)]
                li = carry[0]
                accs = carry[1:]
                first = idv[0]
                prev = lax.gather(
                    idv, shift_idx[:, None],
                    lax.GatherDimensionNumbers(
                        offset_dims=(), collapsed_slice_dims=(0,),
                        start_index_map=(0,)),
                    slice_sizes=(1,),
                    mode=lax.GatherScatterMode.PROMISE_IN_BOUNDS)
                m_int = (idv != prev) & (iota16 > 0)
                b_int = plsc.all_reduce_population_count(m_int)[0]
                m0 = (first - slo) != li
                nb = b_int + jnp.where(m0, 1, 0)

                n8 = jnp.where(nb == 0, 2, 0)

                def fast8(b, accs):
                    rb = r0 + b * 8
                    for i in range(8):
                        accs = plain_row(rb + i, accs)
                    return accs

                accs = lax.fori_loop(0, n8, fast8, accs)

                p = jnp.where(m0, 0, plsc.all_reduce_ffs(m_int)[0])
                e1 = jnp.where(nb == 1, p, 0)
                accs = lax.fori_loop(r0, r0 + e1, plain_row, accs)

                @pl.when((nb == 1) & (li >= 0))
                def _():
                    for j in range(NJ):
                        sl = pl.ds(j * N_LANE, N_LANE)
                        acc[li, sl] = jnp.maximum(acc[li, sl], accs[j])

                accs = tuple(jnp.where(nb == 1, neg_inf, accs[j])
                             for j in range(NJ))
                s2 = jnp.where(nb == 1, p, UNROLL)
                accs = lax.fori_loop(r0 + s2, r0 + UNROLL, plain_row, accs)
                li = jnp.where(nb == 1, idv[15] - slo, li)

                def slow16(_, carry):
                    for i in range(UNROLL):
                        carry = step(xbuf, r0 + i, idv[i] - slo, carry)
                    return carry

                nslow = jnp.where(nb >= 2, 1, 0)
                return lax.fori_loop(0, nslow, slow16, (li,) + accs)

            carry = lax.fori_loop(0, nq, group_body, carry)
            return lax.fori_loop(lo_al + nq * UNROLL, hi_rel, row_body, carry)

        n = c_hi - c_lo

        @pl.when(n > 0)
        def _():
            start(c_lo, xbuf0, idbuf0, sem0)

        @pl.loop(0, SEG_PER_W)
        def _(r):
            for j in range(NJ):
                acc[r, pl.ds(j * N_LANE, N_LANE)] = neg_inf

        def pair_body(kk, carry):
            c0 = c_lo + 2 * kk
            wait(c0, xbuf0, idbuf0, sem0)

            @pl.when(c0 + 1 < c_hi)
            def _():
                start(c0 + 1, xbuf1, idbuf1, sem1)

            carry = process(c0, xbuf0, idbuf0, carry)

            @pl.when(c0 + 1 < c_hi)
            def _():
                wait(c0 + 1, xbuf1, idbuf1, sem1)

            @pl.when(c0 + 2 < c_hi)
            def _():
                start(c0 + 2, xbuf0, idbuf0, sem0)

            carry = process(c0 + 1, xbuf1, idbuf1, carry)
            return carry

        init = (jnp.int32(-1),) + tuple(neg_inf for _ in range(NJ))
        final = lax.fori_loop(0, lax.div(n + 1, 2), pair_body, init)
        fli = final[0]

        @pl.when(fli >= 0)
        def _():
            for j in range(NJ):
                sl = pl.ds(j * N_LANE, N_LANE)
                acc[fli, sl] = jnp.maximum(acc[fli, sl], final[1 + j])

        pltpu.sync_copy(acc, out_hbm.at[pl.ds(slo, SEG_PER_W)])

    return sc_kernel(x, ids32, row_bounds)


def kernel(x, segment_ids):
    ids32 = segment_ids.astype(jnp.int32)
    row_bounds = _row_bounds(ids32)
    out_pad = _sc_segment_max(x, ids32, row_bounds)
    return out_pad[:N_SEG]
